# baseline (speedup 1.0000x reference)
import functools

import jax
import jax.numpy as jnp
from jax import lax
from jax.experimental import pallas as pl
from jax.experimental.pallas import tpu as pltpu
from jax.experimental.pallas import tpu_sc as plsc

_NUM_CORES = 2
_NUM_SUBCORES = 16
_NUM_WORKERS = _NUM_CORES * _NUM_SUBCORES


@functools.partial(jax.jit, static_argnames=("n_tables", "vocab", "dim", "dense_dim"))
def _merged_gather(idx_flat, dense, tab_flat, *, n_tables, vocab, dim, dense_dim):
    batch = idx_flat.shape[1]
    pad_dim = (n_tables + 1) * dim
    bpw = batch // _NUM_WORKERS

    mesh = plsc.VectorSubcoreMesh(
        core_axis_name="c",
        subcore_axis_name="s",
        num_cores=_NUM_CORES,
        num_subcores=_NUM_SUBCORES,
    )

    @functools.partial(
        pl.kernel,
        out_type=jax.ShapeDtypeStruct((batch, pad_dim), jnp.float32),
        mesh=mesh,
        compiler_params=pltpu.CompilerParams(
            use_tc_tiling_on_sc=True,
            needs_layout_passes=False,
            disable_bounds_checks=True,
        ),
        scratch_types=[
            pltpu.VMEM((n_tables, bpw), jnp.int32),
            pltpu.VMEM((2, bpw, dim), jnp.float32),
            pltpu.VMEM((bpw, dim), jnp.float32),
            pltpu.SemaphoreType.DMA,
            pltpu.SemaphoreType.DMA,
            pltpu.SemaphoreType.DMA,
            pltpu.SemaphoreType.DMA,
            pltpu.SemaphoreType.DMA,
        ],
    )
    def body(
        idx_hbm, dense_hbm, tab_hbm, out_hbm, idx_v, rows_v, dense_v, g0, g1, w0, w1, wd
    ):
        gsem = [g0, g1]
        wsem = [w0, w1]
        wid = lax.axis_index("s") * _NUM_CORES + lax.axis_index("c")
        b0 = wid * bpw

        pltpu.sync_copy(idx_hbm.at[:, pl.ds(b0, bpw)], idx_v)

        pltpu.sync_copy(dense_hbm.at[pl.ds(b0, bpw)], dense_v)
        pltpu.async_copy(dense_v, out_hbm.at[pl.ds(b0, bpw), pl.ds(0, dim)], wd)

        def fill(t, buf):
            pltpu.async_copy(tab_hbm.at[idx_v.at[t]], rows_v.at[buf], gsem[buf])

        def wait_write(buf):
            pltpu.make_async_copy(
                rows_v.at[buf],
                out_hbm.at[pl.ds(b0, bpw), pl.ds(0, dim)],
                wsem[buf],
            ).wait()

        def drain(t, buf):
            pltpu.make_async_copy(
                tab_hbm.at[idx_v.at[0]], rows_v.at[buf], gsem[buf]
            ).wait()
            pltpu.async_copy(
                rows_v.at[buf],
                out_hbm.at[pl.ds(b0, bpw), pl.ds((t + 1) * dim, dim)],
                wsem[buf],
            )

        fill(0, 0)
        fill(1, 1)

        @pl.loop(0, n_tables)
        def per_table(t):
            parity = lax.rem(t, 2)

            def step(buf):
                drain(t, buf)

                @pl.when(t + 2 < n_tables)
                def _refill():
                    wait_write(buf)
                    fill(t + 2, buf)

            @pl.when(parity == 0)
            def _even():
                step(0)

            @pl.when(parity == 1)
            def _odd():
                step(1)

        wait_write((n_tables - 2) % 2)
        wait_write((n_tables - 1) % 2)
        pltpu.make_async_copy(
            dense_v, out_hbm.at[pl.ds(b0, bpw), pl.ds(0, dim)], wd
        ).wait()

    return body(idx_flat, dense, tab_flat)


def kernel(indices, offsets, dense, tables):
    del offsets
    n_tables, batch = indices.shape
    _, vocab, dim = tables.shape
    dense_dim = dense.shape[1]
    idx_flat = indices + (jnp.arange(n_tables, dtype=jnp.int32) * vocab)[:, None]
    tab_flat = tables.reshape(n_tables * vocab, dim)
    dense_pad = jnp.pad(dense, ((0, 0), (dim - dense_dim, 0)))
    padded = _merged_gather(
        idx_flat,
        dense_pad,
        tab_flat,
        n_tables=n_tables,
        vocab=vocab,
        dim=dim,
        dense_dim=dense_dim,
    )
    return padded[:, dim - dense_dim :]
